# baseline (speedup 1.0000x reference)
import functools

import jax
import jax.numpy as jnp
from jax import lax
from jax.experimental import pallas as pl
from jax.experimental.pallas import tpu as pltpu
from jax.experimental.pallas import tpu_sc as plsc

DIM = 64
LANES = 16
NC, NS = 2, 16
NW = NC * NS
CHUNK = 128
NSLOT = 2

PROJ_BLK = 2000


def _project_kernel(w0, w1, w2, m0, m1, m2, bb, pa, pb):
    c0 = jnp.dot(w0[...], m0[...], preferred_element_type=jnp.float32)
    c1 = jnp.dot(w1[...], m1[...], preferred_element_type=jnp.float32)
    c2 = jnp.dot(w2[...], m2[...], preferred_element_type=jnp.float32)
    third = bb[...]
    pa[...] = jnp.concatenate([c0 + third, c1 + third], axis=1)
    pb[...] = jnp.concatenate([c2 + third, c2 + third], axis=1)


def _project(W0, W1, W2, M0, M1, M2, bb):
    vocab = W0.shape[0]
    assert vocab % PROJ_BLK == 0
    nblk = vocab // PROJ_BLK
    row_spec = pl.BlockSpec((PROJ_BLK, DIM), lambda i: (i, 0))
    out_spec = pl.BlockSpec((PROJ_BLK, 2 * DIM), lambda i: (i, 0))
    full_spec = pl.BlockSpec((DIM, DIM), lambda i: (0, 0))
    bias_spec = pl.BlockSpec((1, DIM), lambda i: (0, 0))
    return pl.pallas_call(
        _project_kernel,
        grid=(nblk,),
        in_specs=[row_spec, row_spec, row_spec,
                  full_spec, full_spec, full_spec, bias_spec],
        out_specs=[out_spec, out_spec],
        out_shape=[jax.ShapeDtypeStruct((vocab, 2 * DIM), jnp.float32)] * 2,
    )(W0, W1, W2, M0, M1, M2, bb)


def _make_gather_sum(n_tokens):
    n_per_w = n_tokens // NW
    n_chunks = n_per_w // CHUNK
    assert n_per_w % CHUNK == 0
    mesh = plsc.VectorSubcoreMesh(core_axis_name="c", subcore_axis_name="s")

    @functools.partial(
        pl.kernel,
        mesh=mesh,
        out_type=jax.ShapeDtypeStruct((n_tokens // 2, 2 * DIM), jnp.float32),
        scratch_types=[
            pltpu.VMEM((3 * n_per_w,), jnp.int32),
            pltpu.VMEM((NSLOT, CHUNK, 2 * DIM), jnp.float32),
            pltpu.VMEM((NSLOT, CHUNK, 2 * DIM), jnp.float32),
            pltpu.VMEM((NSLOT, CHUNK, 2 * DIM), jnp.float32),
            pltpu.VMEM((CHUNK // 2, 2 * DIM), jnp.float32),
            pltpu.SemaphoreType.DMA((NSLOT,)),
        ],
    )
    def gather_sum(pa_hbm, pb_hbm, it_hbm,
                   out_hbm, i_v, r0_v, r1_v, r2_v, o_v, sems):
        wid = lax.axis_index("s") * NC + lax.axis_index("c")
        base = wid * n_per_w

        for k in range(3):
            src = pl.ds(pl.multiple_of(k * n_tokens + base, CHUNK), n_per_w)
            dst = pl.ds(k * n_per_w, n_per_w)
            pltpu.sync_copy(it_hbm.at[src], i_v.at[dst])

        def start(ch, slot):
            off = pl.multiple_of(ch * CHUNK, CHUNK)
            for k, rv in ((0, r0_v), (1, r1_v), (2, r2_v)):
                sl = pl.ds(k * n_per_w + off, CHUNK)
                tab = pb_hbm if k == 2 else pa_hbm
                pltpu.async_copy(
                    tab.at[i_v.at[sl]], rv.at[slot], sems.at[slot])

        def finish(ch, slot):
            for rv in (r0_v, r1_v, r2_v):
                pltpu.make_async_copy(
                    pa_hbm.at[i_v.at[pl.ds(0, CHUNK)]], rv.at[slot],
                    sems.at[slot]).wait()
            r0s, r1s, r2s = r0_v.at[slot], r1_v.at[slot], r2_v.at[slot]

            def pair_body(u, _):
                for r in range(2):
                    t = 2 * u + r
                    for k in range(DIM // LANES):
                        src = pl.ds(k * LANES, LANES)
                        hi = pl.ds(DIM + k * LANES, LANES)
                        v = r0s[t, src] + r1s[t, hi] + r2s[t, src]
                        o_v[u, pl.ds(r * DIM + k * LANES, LANES)] = (
                            jnp.maximum(v, 0.0))
                return 0

            lax.fori_loop(0, CHUNK // 2, pair_body, 0, unroll=2)
            tok0 = pl.multiple_of(base + ch * CHUNK, CHUNK)
            row0 = pl.multiple_of(tok0 // 2, CHUNK // 2)
            pltpu.sync_copy(o_v, out_hbm.at[pl.ds(row0, CHUNK // 2)])

        n_pairs = n_chunks // NSLOT
        start(0, 0)

        def pair_of_chunks(g, _):
            start(NSLOT * g + 1, 1)
            finish(NSLOT * g, 0)

            @pl.when(g + 1 < n_pairs)
            def _():
                start(NSLOT * g + 2, 0)

            finish(NSLOT * g + 1, 1)
            return 0

        lax.fori_loop(0, n_pairs, pair_of_chunks, 0)

    return gather_sum


def kernel(input, W0, W1, W2, Wm, b):
    seq, batch, _ = input.shape
    n = seq * batch
    it = jnp.transpose(input.reshape(n, 3).astype(jnp.int32)).reshape(3 * n)
    M0 = Wm[:, 0 * DIM:1 * DIM].T
    M1 = Wm[:, 1 * DIM:2 * DIM].T
    M2 = Wm[:, 2 * DIM:3 * DIM].T
    bb = (b * (1.0 / 3.0)).reshape(1, DIM)
    pa, pb = _project(W0, W1, W2, M0, M1, M2, bb)
    out = _make_gather_sum(n)(pa, pb, it)
    return out.reshape(seq, batch, DIM)


# speedup vs baseline: 1.0087x; 1.0087x over previous
import functools

import jax
import jax.numpy as jnp
from jax import lax
from jax.experimental import pallas as pl
from jax.experimental.pallas import tpu as pltpu
from jax.experimental.pallas import tpu_sc as plsc

DIM = 64
LANES = 16
NC, NS = 2, 16
NW = NC * NS
CHUNK = 128
NSLOT = 2

PROJ_BLK = 2000


def _project_kernel(w0, w1, w2, m0, m1, m2, bb, pa, pb):
    c0 = jnp.dot(w0[...], m0[...], preferred_element_type=jnp.float32)
    c1 = jnp.dot(w1[...], m1[...], preferred_element_type=jnp.float32)
    c2 = jnp.dot(w2[...], m2[...], preferred_element_type=jnp.float32)
    third = bb[...]
    pa[...] = jnp.concatenate([c0 + third, c1 + third], axis=1)
    pb[...] = jnp.concatenate([c2 + third, c2 + third], axis=1)


def _project(W0, W1, W2, M0, M1, M2, bb):
    vocab = W0.shape[0]
    assert vocab % PROJ_BLK == 0
    nblk = vocab // PROJ_BLK
    row_spec = pl.BlockSpec((PROJ_BLK, DIM), lambda i: (i, 0))
    out_spec = pl.BlockSpec((PROJ_BLK, 2 * DIM), lambda i: (i, 0))
    full_spec = pl.BlockSpec((DIM, DIM), lambda i: (0, 0))
    bias_spec = pl.BlockSpec((1, DIM), lambda i: (0, 0))
    return pl.pallas_call(
        _project_kernel,
        grid=(nblk,),
        in_specs=[row_spec, row_spec, row_spec,
                  full_spec, full_spec, full_spec, bias_spec],
        out_specs=[out_spec, out_spec],
        out_shape=[jax.ShapeDtypeStruct((vocab, 2 * DIM), jnp.float32)] * 2,
    )(W0, W1, W2, M0, M1, M2, bb)


def _make_gather_sum(n_tokens):
    n_per_w = n_tokens // NW
    n_chunks = n_per_w // CHUNK
    assert n_per_w % CHUNK == 0
    mesh = plsc.VectorSubcoreMesh(core_axis_name="c", subcore_axis_name="s")

    @functools.partial(
        pl.kernel,
        mesh=mesh,
        out_type=jax.ShapeDtypeStruct((n_tokens // 2, 2 * DIM), jnp.float32),
        scratch_types=[
            pltpu.VMEM((3 * n_per_w,), jnp.int32),
            pltpu.VMEM((NSLOT, CHUNK, 2 * DIM), jnp.float32),
            pltpu.VMEM((NSLOT, CHUNK, 2 * DIM), jnp.float32),
            pltpu.VMEM((NSLOT, CHUNK, 2 * DIM), jnp.float32),
            pltpu.VMEM((CHUNK // 2, 2 * DIM), jnp.float32),
            pltpu.SemaphoreType.DMA((NSLOT,)),
        ],
    )
    def gather_sum(pa_hbm, pb_hbm, it_hbm,
                   out_hbm, i_v, r0_v, r1_v, r2_v, o_v, sems):
        wid = lax.axis_index("s") * NC + lax.axis_index("c")
        base = wid * n_per_w

        for k in range(3):
            src = pl.ds(pl.multiple_of(k * n_tokens + base, CHUNK), n_per_w)
            dst = pl.ds(k * n_per_w, n_per_w)
            pltpu.sync_copy(it_hbm.at[src], i_v.at[dst])

        def start(ch, slot):
            off = pl.multiple_of(ch * CHUNK, CHUNK)
            for k, rv in ((0, r0_v), (1, r1_v), (2, r2_v)):
                sl = pl.ds(k * n_per_w + off, CHUNK)
                tab = pb_hbm if k == 2 else pa_hbm
                pltpu.async_copy(
                    tab.at[i_v.at[sl]], rv.at[slot], sems.at[slot])

        def finish(ch, slot):
            for rv in (r0_v, r1_v, r2_v):
                pltpu.make_async_copy(
                    pa_hbm.at[i_v.at[pl.ds(0, CHUNK)]], rv.at[slot],
                    sems.at[slot]).wait()
            r0s, r1s, r2s = r0_v.at[slot], r1_v.at[slot], r2_v.at[slot]

            def pair_body(u, _):
                for r in range(2):
                    t = 2 * u + r
                    for k in range(DIM // LANES):
                        src = pl.ds(k * LANES, LANES)
                        hi = pl.ds(DIM + k * LANES, LANES)
                        v = r0s[t, src] + r1s[t, hi] + r2s[t, src]
                        o_v[u, pl.ds(r * DIM + k * LANES, LANES)] = (
                            jnp.maximum(v, 0.0))
                return 0

            lax.fori_loop(0, CHUNK // 2, pair_body, 0, unroll=2)
            tok0 = pl.multiple_of(base + ch * CHUNK, CHUNK)
            row0 = pl.multiple_of(tok0 // 2, CHUNK // 2)
            pltpu.sync_copy(o_v, out_hbm.at[pl.ds(row0, CHUNK // 2)])

        n_pairs = n_chunks // NSLOT
        start(0, 0)

        def pair_of_chunks(g, _):
            start(NSLOT * g + 1, 1)
            finish(NSLOT * g, 0)

            @pl.when(g + 1 < n_pairs)
            def _():
                start(NSLOT * g + 2, 0)

            finish(NSLOT * g + 1, 1)
            return 0

        lax.fori_loop(0, n_pairs, pair_of_chunks, 0)

    return gather_sum


def kernel(input, W0, W1, W2, Wm, b):
    seq, batch, _ = input.shape
    n = seq * batch
    idx = input.reshape(n, 3).astype(jnp.int32)
    eye = jnp.eye(3, dtype=jnp.int32)
    cols = [jnp.sum(idx * eye[k][None, :], axis=1) for k in range(3)]
    it = jnp.concatenate(cols)
    M0 = Wm[:, 0 * DIM:1 * DIM].T
    M1 = Wm[:, 1 * DIM:2 * DIM].T
    M2 = Wm[:, 2 * DIM:3 * DIM].T
    bb = (b * (1.0 / 3.0)).reshape(1, DIM)
    pa, pb = _project(W0, W1, W2, M0, M1, M2, bb)
    out = _make_gather_sum(n)(pa, pb, it)
    return out.reshape(seq, batch, DIM)


# speedup vs baseline: 1.2533x; 1.2424x over previous
import functools

import jax
import jax.numpy as jnp
from jax import lax
from jax.experimental import pallas as pl
from jax.experimental.pallas import tpu as pltpu
from jax.experimental.pallas import tpu_sc as plsc

DIM = 64
LANES = 16
NC, NS = 2, 16
NW = NC * NS
CHUNK = 128
NSLOT = 2

PROJ_BLK = 2048


def _project_kernel(w0t, w1t, w2t, m0, m1, m2, bb, pa, pb):
    cd = (((0,), (0,)), ((), ()))
    c0 = lax.dot_general(w0t[...], m0[...], cd,
                         preferred_element_type=jnp.float32)
    c1 = lax.dot_general(w1t[...], m1[...], cd,
                         preferred_element_type=jnp.float32)
    c2 = lax.dot_general(w2t[...], m2[...], cd,
                         preferred_element_type=jnp.float32)
    third = bb[...]
    pa[...] = jnp.concatenate([c0 + third, c1 + third], axis=1)
    pb[...] = jnp.concatenate([c2 + third, c2 + third], axis=1)


def _project(W0t, W1t, W2t, M0, M1, M2, bb):
    vocab = W0t.shape[1]
    nblk = (vocab + PROJ_BLK - 1) // PROJ_BLK
    wt_spec = pl.BlockSpec((DIM, PROJ_BLK), lambda i: (0, i))
    out_spec = pl.BlockSpec((PROJ_BLK, 2 * DIM), lambda i: (i, 0))
    full_spec = pl.BlockSpec((DIM, DIM), lambda i: (0, 0))
    bias_spec = pl.BlockSpec((1, DIM), lambda i: (0, 0))
    return pl.pallas_call(
        _project_kernel,
        grid=(nblk,),
        in_specs=[wt_spec, wt_spec, wt_spec,
                  full_spec, full_spec, full_spec, bias_spec],
        out_specs=[out_spec, out_spec],
        out_shape=[jax.ShapeDtypeStruct((vocab, 2 * DIM), jnp.float32)] * 2,
    )(W0t, W1t, W2t, M0, M1, M2, bb)


def _make_gather_sum(n_tokens):
    n_per_w = n_tokens // NW
    n_chunks = n_per_w // CHUNK
    assert n_per_w % CHUNK == 0
    mesh = plsc.VectorSubcoreMesh(core_axis_name="c", subcore_axis_name="s")

    @functools.partial(
        pl.kernel,
        mesh=mesh,
        out_type=jax.ShapeDtypeStruct((n_tokens // 2, 2 * DIM), jnp.float32),
        scratch_types=[
            pltpu.VMEM((3 * n_per_w,), jnp.int32),
            pltpu.VMEM((NSLOT, CHUNK, 2 * DIM), jnp.float32),
            pltpu.VMEM((NSLOT, CHUNK, 2 * DIM), jnp.float32),
            pltpu.VMEM((NSLOT, CHUNK, 2 * DIM), jnp.float32),
            pltpu.VMEM((CHUNK // 2, 2 * DIM), jnp.float32),
            pltpu.SemaphoreType.DMA((NSLOT,)),
        ],
    )
    def gather_sum(pa_hbm, pb_hbm, it_hbm,
                   out_hbm, i_v, r0_v, r1_v, r2_v, o_v, sems):
        wid = lax.axis_index("s") * NC + lax.axis_index("c")
        base = wid * n_per_w

        for k in range(3):
            src = pl.ds(pl.multiple_of(k * n_tokens + base, CHUNK), n_per_w)
            dst = pl.ds(k * n_per_w, n_per_w)
            pltpu.sync_copy(it_hbm.at[src], i_v.at[dst])

        def start(ch, slot):
            off = pl.multiple_of(ch * CHUNK, CHUNK)
            for k, rv in ((0, r0_v), (1, r1_v), (2, r2_v)):
                sl = pl.ds(k * n_per_w + off, CHUNK)
                tab = pb_hbm if k == 2 else pa_hbm
                pltpu.async_copy(
                    tab.at[i_v.at[sl]], rv.at[slot], sems.at[slot])

        def finish(ch, slot):
            for rv in (r0_v, r1_v, r2_v):
                pltpu.make_async_copy(
                    pa_hbm.at[i_v.at[pl.ds(0, CHUNK)]], rv.at[slot],
                    sems.at[slot]).wait()
            r0s, r1s, r2s = r0_v.at[slot], r1_v.at[slot], r2_v.at[slot]

            def pair_body(u, _):
                for r in range(2):
                    t = 2 * u + r
                    for k in range(DIM // LANES):
                        src = pl.ds(k * LANES, LANES)
                        hi = pl.ds(DIM + k * LANES, LANES)
                        v = r0s[t, src] + r1s[t, hi] + r2s[t, src]
                        o_v[u, pl.ds(r * DIM + k * LANES, LANES)] = (
                            jnp.maximum(v, 0.0))
                return 0

            lax.fori_loop(0, CHUNK // 2, pair_body, 0, unroll=2)
            tok0 = pl.multiple_of(base + ch * CHUNK, CHUNK)
            row0 = pl.multiple_of(tok0 // 2, CHUNK // 2)
            pltpu.sync_copy(o_v, out_hbm.at[pl.ds(row0, CHUNK // 2)])

        n_pairs = n_chunks // NSLOT
        start(0, 0)

        def pair_of_chunks(g, _):
            start(NSLOT * g + 1, 1)
            finish(NSLOT * g, 0)

            @pl.when(g + 1 < n_pairs)
            def _():
                start(NSLOT * g + 2, 0)

            finish(NSLOT * g + 1, 1)
            return 0

        lax.fori_loop(0, n_pairs, pair_of_chunks, 0)

    return gather_sum


def kernel(input, W0, W1, W2, Wm, b):
    seq, batch, _ = input.shape
    n = seq * batch
    idx = input.reshape(n, 3).astype(jnp.int32)
    eye = jnp.eye(3, dtype=jnp.int32)
    cols = [jnp.sum(idx * eye[k][None, :], axis=1) for k in range(3)]
    it = jnp.concatenate(cols)
    M0 = Wm[:, 0 * DIM:1 * DIM].T
    M1 = Wm[:, 1 * DIM:2 * DIM].T
    M2 = Wm[:, 2 * DIM:3 * DIM].T
    bb = (b * (1.0 / 3.0)).reshape(1, DIM)
    pa, pb = _project(jnp.transpose(W0), jnp.transpose(W1), jnp.transpose(W2),
                      M0, M1, M2, bb)
    out = _make_gather_sum(n)(pa, pb, it)
    return out.reshape(seq, batch, DIM)
